# baseline (speedup 1.0000x reference)
import functools

import jax
import jax.numpy as jnp
from jax import lax
from jax.experimental import pallas as pl
from jax.experimental.pallas import tpu as pltpu
from jax.experimental.pallas import tpu_sc as plsc

N_V = 1000
N_D = 64
N_B = 4096
N_H = 200
NUM_CORES = 2
NUM_SUBCORES = 16
NW = NUM_CORES * NUM_SUBCORES
BPT = N_B // NW
HC = 2
NCH = N_H // HC
OUT_ELEMS = N_H * 8 * NW * 8 * 128


def _sc_embed(table_flat, idx_flat):
    mesh = plsc.VectorSubcoreMesh(core_axis_name="c", subcore_axis_name="s")

    @functools.partial(
        pl.kernel,
        out_type=jax.ShapeDtypeStruct((OUT_ELEMS,), jnp.float32),
        mesh=mesh,
        scratch_types=[
            pltpu.VMEM((N_V * N_D,), jnp.float32),
            pltpu.VMEM((BPT * N_H,), jnp.int32),
            pltpu.VMEM((HC * 8 * 8 * 128,), jnp.float32),
            pltpu.VMEM((HC * 8 * 8 * 128,), jnp.float32),
            pltpu.SemaphoreType.DMA,
            pltpu.SemaphoreType.DMA,
        ],
        compiler_params=pltpu.CompilerParams(use_tc_tiling_on_sc=False, needs_layout_passes=False),
    )
    def k(table_hbm, idx_hbm, out_hbm, table_v, idx_v, ob0, ob1, os0, os1):
        wid = lax.axis_index("s") * NUM_CORES + lax.axis_index("c")
        pltpu.sync_copy(table_hbm, table_v)
        pltpu.sync_copy(idx_hbm.at[pl.ds(wid * BPT * N_H, BPT * N_H)], idx_v)

        obuf = (ob0, ob1)
        osem = (os0, os1)
        iota = lax.iota(jnp.int32, 16)
        iota_h = iota * N_H

        def build_chunk(hc, b):
            bases = []
            for hh in range(HC):
                h = hc * HC + hh
                for bl0 in range(8):
                    pos = iota_h + (bl0 * 16 * N_H + h)
                    bases.append(plsc.load_gather(idx_v, [pos]))

            def dt_body(dt, carry):
                for dp in range(4):
                    vals = []
                    for d_h in range(2):
                        d_s = dp * 2 + d_h
                        t = (dt * 8 + d_s) * N_V
                        tab_d = table_v.at[pl.ds(t, N_V)]
                        vals.append([plsc.load_gather(tab_d, [bases[g]])
                                     for g in range(16)])
                    for d_h in range(2):
                        d_s = dp * 2 + d_h
                        for g in range(16):
                            off = (dt * 1024
                                   + (g // 8) * 8192 + d_s * 128 + (g % 8) * 16)
                            obuf[b][pl.ds(off, 16)] = vals[d_h][g]
                return carry

            lax.fori_loop(0, 8, dt_body, 0)

        def out_descs(hc, b):
            descs = []
            for hh in range(HC):
                h = hc * HC + hh
                for dt in range(8):
                    dst_off = ((h * 8 + dt) * NW + wid) * 1024
                    src_off = (hh * 8 + dt) * 1024
                    descs.append(pltpu.make_async_copy(
                        obuf[b].at[pl.ds(src_off, 1024)],
                        out_hbm.at[pl.ds(dst_off, 1024)],
                        osem[b]))
            return descs

        def fire(hc, b):
            for dsc in out_descs(hc, b):
                dsc.start()

        def drain(hc, b):
            for dsc in out_descs(hc, b):
                dsc.wait()

        build_chunk(0, 0)
        fire(0, 0)
        build_chunk(1, 1)
        fire(1, 1)

        def body(j, carry):
            for b in range(2):
                hc = 2 * j + 2 + b
                drain(hc, b)
                build_chunk(hc, b)
                fire(hc, b)
            return carry

        lax.fori_loop(0, (NCH - 2) // 2, body, 0)
        drain(NCH - 2, 0)
        drain(NCH - 1, 1)

    return k(table_flat, idx_flat)


def kernel(input, table):
    idx_flat = input.reshape(-1).astype(jnp.int32)
    out_flat = _sc_embed(table.astype(jnp.float32).T.reshape(-1), idx_flat)
    out5 = out_flat.reshape(N_H, 8, NW, 8, 128)
    return jnp.transpose(out5, (2, 4, 0, 1, 3)).reshape(N_B, N_H, N_D)


# speedup vs baseline: 1.0497x; 1.0497x over previous
import functools

import jax
import jax.numpy as jnp
from jax import lax
from jax.experimental import pallas as pl
from jax.experimental.pallas import tpu as pltpu
from jax.experimental.pallas import tpu_sc as plsc

N_V = 1000
N_D = 64
N_B = 4096
N_H = 200
NUM_CORES = 2
NUM_SUBCORES = 16
NW = NUM_CORES * NUM_SUBCORES
BPT = N_B // NW
HC = 2
NCH = N_H // HC
OUT_ELEMS = N_H * 8 * NW * 8 * 128


def _sc_embed(table_flat, idx_flat):
    mesh = plsc.VectorSubcoreMesh(core_axis_name="c", subcore_axis_name="s")

    @functools.partial(
        pl.kernel,
        out_type=jax.ShapeDtypeStruct((N_H, 8, NW, 8, 128), jnp.float32),
        mesh=mesh,
        scratch_types=[
            pltpu.VMEM((N_V * N_D,), jnp.float32),
            pltpu.VMEM((BPT * N_H,), jnp.int32),
            pltpu.VMEM((HC, 8, 1, 8, 128), jnp.float32),
            pltpu.VMEM((HC, 8, 1, 8, 128), jnp.float32),
            pltpu.SemaphoreType.DMA,
            pltpu.SemaphoreType.DMA,
        ],
        compiler_params=pltpu.CompilerParams(use_tc_tiling_on_sc=False, needs_layout_passes=False),
    )
    def k(table_hbm, idx_hbm, out_hbm, table_v, idx_v, ob0, ob1, os0, os1):
        wid = lax.axis_index("s") * NUM_CORES + lax.axis_index("c")
        pltpu.sync_copy(table_hbm, table_v)
        pltpu.sync_copy(idx_hbm.at[pl.ds(wid * BPT * N_H, BPT * N_H)], idx_v)

        obuf = (ob0, ob1)
        osem = (os0, os1)
        iota = lax.iota(jnp.int32, 16)
        iota_h = iota * N_H

        def build_chunk(hc, b):
            bases = []
            for hh in range(HC):
                h = hc * HC + hh
                for bl0 in range(8):
                    pos = iota_h + (bl0 * 16 * N_H + h)
                    bases.append(plsc.load_gather(idx_v, [pos]))

            def dt_body(dt, carry):
                for d_s in range(8):
                    t = (dt * 8 + d_s) * N_V
                    tab_d = table_v.at[pl.ds(t, N_V)]
                    vals = [plsc.load_gather(tab_d, [bases[g]])
                            for g in range(16)]
                    for g in range(16):
                        obuf[b][g // 8, dt, 0, d_s,
                                pl.ds((g % 8) * 16, 16)] = vals[g]
                return carry

            lax.fori_loop(0, 8, dt_body, 0)

        def out_desc(hc, b):
            return pltpu.make_async_copy(
                obuf[b],
                out_hbm.at[pl.ds(hc * HC, HC), pl.ds(0, 8), pl.ds(wid, 1)],
                osem[b])

        def fire(hc, b):
            out_desc(hc, b).start()

        def drain(hc, b):
            out_desc(hc, b).wait()

        build_chunk(0, 0)
        fire(0, 0)
        build_chunk(1, 1)
        fire(1, 1)

        def body(j, carry):
            for b in range(2):
                hc = 2 * j + 2 + b
                drain(hc, b)
                build_chunk(hc, b)
                fire(hc, b)
            return carry

        lax.fori_loop(0, (NCH - 2) // 2, body, 0)
        drain(NCH - 2, 0)
        drain(NCH - 1, 1)

    return k(table_flat, idx_flat)


def kernel(input, table):
    idx_flat = input.reshape(-1).astype(jnp.int32)
    out_flat = _sc_embed(table.astype(jnp.float32).T.reshape(-1), idx_flat)
    out5 = out_flat.reshape(N_H, 8, NW, 8, 128)
    return jnp.transpose(out5, (2, 4, 0, 1, 3)).reshape(N_B, N_H, N_D)


# speedup vs baseline: 1.0527x; 1.0028x over previous
import functools

import jax
import jax.numpy as jnp
from jax import lax
from jax.experimental import pallas as pl
from jax.experimental.pallas import tpu as pltpu
from jax.experimental.pallas import tpu_sc as plsc

N_V = 1000
N_D = 64
N_B = 4096
N_H = 200
NUM_CORES = 2
NUM_SUBCORES = 16
NW = NUM_CORES * NUM_SUBCORES
BPT = N_B // NW
HC = 2
NCH = N_H // HC
OUT_ELEMS = N_H * 8 * NW * 8 * 128


def _sc_embed(table_flat, idx_flat):
    mesh = plsc.VectorSubcoreMesh(core_axis_name="c", subcore_axis_name="s")

    @functools.partial(
        pl.kernel,
        out_type=jax.ShapeDtypeStruct((N_H, 8, NW, 8, 128), jnp.float32),
        mesh=mesh,
        scratch_types=[
            pltpu.VMEM((N_V * N_D,), jnp.float32),
            pltpu.VMEM((BPT * N_H,), jnp.int32),
            pltpu.VMEM((HC, 8, 1, 8, 128), jnp.float32),
            pltpu.VMEM((HC, 8, 1, 8, 128), jnp.float32),
            pltpu.SemaphoreType.DMA,
            pltpu.SemaphoreType.DMA,
            pltpu.SemaphoreType.DMA,
        ],
        compiler_params=pltpu.CompilerParams(use_tc_tiling_on_sc=False, needs_layout_passes=False),
    )
    def k(table_hbm, idx_hbm, out_hbm, table_v, idx_v, ob0, ob1, os0, os1,
          ssem):
        wid = lax.axis_index("s") * NUM_CORES + lax.axis_index("c")
        tcopy = pltpu.async_copy(table_hbm, table_v, ssem)
        icopy = pltpu.async_copy(
            idx_hbm.at[pl.ds(wid * BPT * N_H, BPT * N_H)], idx_v, ssem)
        tcopy.wait()
        icopy.wait()

        obuf = (ob0, ob1)
        osem = (os0, os1)
        iota = lax.iota(jnp.int32, 16)
        iota_h = iota * N_H

        def build_chunk(hc, b):
            bases = []
            for hh in range(HC):
                h = hc * HC + hh
                for bl0 in range(8):
                    pos = iota_h + (bl0 * 16 * N_H + h)
                    bases.append(plsc.load_gather(idx_v, [pos]))

            def dt_body(dt, carry):
                for d_s in range(8):
                    t = (dt * 8 + d_s) * N_V
                    tab_d = table_v.at[pl.ds(t, N_V)]
                    vals = [plsc.load_gather(tab_d, [bases[g]])
                            for g in range(16)]
                    for g in range(16):
                        obuf[b][g // 8, dt, 0, d_s,
                                pl.ds((g % 8) * 16, 16)] = vals[g]
                return carry

            lax.fori_loop(0, 8, dt_body, 0)

        def out_desc(hc, b):
            return pltpu.make_async_copy(
                obuf[b],
                out_hbm.at[pl.ds(hc * HC, HC), pl.ds(0, 8), pl.ds(wid, 1)],
                osem[b])

        def fire(hc, b):
            out_desc(hc, b).start()

        def drain(hc, b):
            out_desc(hc, b).wait()

        build_chunk(0, 0)
        fire(0, 0)
        build_chunk(1, 1)
        fire(1, 1)

        def body(j, carry):
            for b in range(2):
                hc = 2 * j + 2 + b
                drain(hc, b)
                build_chunk(hc, b)
                fire(hc, b)
            return carry

        lax.fori_loop(0, (NCH - 2) // 2, body, 0)
        drain(NCH - 2, 0)
        drain(NCH - 1, 1)

    return k(table_flat, idx_flat)


def kernel(input, table):
    idx_flat = input.reshape(-1).astype(jnp.int32)
    out_flat = _sc_embed(table.astype(jnp.float32).T.reshape(-1), idx_flat)
    out5 = out_flat.reshape(N_H, 8, NW, 8, 128)
    return jnp.transpose(out5, (2, 4, 0, 1, 3)).reshape(N_B, N_H, N_D)


# speedup vs baseline: 1.0528x; 1.0001x over previous
import functools

import jax
import jax.numpy as jnp
from jax import lax
from jax.experimental import pallas as pl
from jax.experimental.pallas import tpu as pltpu
from jax.experimental.pallas import tpu_sc as plsc

N_V = 1000
N_D = 64
N_B = 4096
N_H = 200
NUM_CORES = 2
NUM_SUBCORES = 16
NW = NUM_CORES * NUM_SUBCORES
BPT = N_B // NW
HC = 2
NCH = N_H // HC


def _sc_embed(table_flat, idx_flat):
    mesh = plsc.VectorSubcoreMesh(core_axis_name="c", subcore_axis_name="s")

    @functools.partial(
        pl.kernel,
        out_type=jax.ShapeDtypeStruct((N_H, 8, NW, 8, 128), jnp.float32),
        mesh=mesh,
        scratch_types=[
            pltpu.VMEM((N_V * N_D,), jnp.float32),
            pltpu.VMEM((BPT * N_H,), jnp.int32),
            pltpu.VMEM((HC, 8, 1, 8, 128), jnp.float32),
            pltpu.VMEM((HC, 8, 1, 8, 128), jnp.float32),
            pltpu.SemaphoreType.DMA,
            pltpu.SemaphoreType.DMA,
            pltpu.SemaphoreType.DMA,
        ],
        compiler_params=pltpu.CompilerParams(use_tc_tiling_on_sc=False, needs_layout_passes=False),
    )
    def k(table_hbm, idx_hbm, out_hbm, table_v, idx_v, ob0, ob1, os0, os1,
          ssem):
        wid = lax.axis_index("s") * NUM_CORES + lax.axis_index("c")
        tcopy = pltpu.async_copy(table_hbm, table_v, ssem)
        icopy = pltpu.async_copy(
            idx_hbm.at[pl.ds(wid * BPT * N_H, BPT * N_H)], idx_v, ssem)
        tcopy.wait()
        icopy.wait()

        obuf = (ob0, ob1)
        osem = (os0, os1)
        iota = lax.iota(jnp.int32, 16)
        iota_h = iota * N_H

        def build_chunk(hc, b):
            bases = []
            for hh in range(HC):
                h = hc * HC + hh
                for bl0 in range(8):
                    pos = iota_h + (bl0 * 16 * N_H + h)
                    bases.append(plsc.load_gather(idx_v, [pos]))

            def dt_body(dt, carry):
                for d_s in range(8):
                    t = (dt * 8 + d_s) * N_V
                    tab_d = table_v.at[pl.ds(t, N_V)]
                    vals = [plsc.load_gather(tab_d, [bases[g]])
                            for g in range(16)]
                    for g in range(16):
                        obuf[b][g // 8, dt, 0, d_s,
                                pl.ds((g % 8) * 16, 16)] = vals[g]
                return carry

            lax.fori_loop(0, 8, dt_body, 0)

        def out_desc(hc, b):
            return pltpu.make_async_copy(
                obuf[b],
                out_hbm.at[pl.ds(hc * HC, HC), pl.ds(0, 8), pl.ds(wid, 1)],
                osem[b])

        def fire(hc, b):
            out_desc(hc, b).start()

        def drain(hc, b):
            out_desc(hc, b).wait()

        build_chunk(0, 0)
        fire(0, 0)
        build_chunk(1, 1)
        fire(1, 1)

        def body(j, carry):
            for b in range(2):
                hc = 2 * j + 2 + b
                drain(hc, b)
                build_chunk(hc, b)
                fire(hc, b)
            return carry

        lax.fori_loop(0, (NCH - 2) // 2, body, 0)
        drain(NCH - 2, 0)
        drain(NCH - 1, 1)

    return k(table_flat, idx_flat)


def kernel(input, table):
    idx_flat = input.reshape(-1).astype(jnp.int32)
    out5 = _sc_embed(table.astype(jnp.float32).T.reshape(-1), idx_flat)
    return jnp.transpose(out5, (2, 4, 0, 1, 3)).reshape(N_B, N_H, N_D)
